# Initial kernel scaffold; baseline (speedup 1.0000x reference)
import jax
import jax.numpy as jnp
from jax.experimental import pallas as pl
from jax.experimental.pallas import tpu as pltpu

B = 4
C = 64
CP = B * C


def _leaky(a):
    return jnp.where(a >= 0, a, 0.2 * a)


def _lane_tile(v):
    return jnp.concatenate([v] * B, axis=1)


def _group_sum(s):
    return sum(s[:, b * C:(b + 1) * C] for b in range(B))


def _bn_from_stats(s1, s2, g, bt, eps, count):
    sum1 = _group_sum(s1)
    sum2 = _group_sum(s2)
    mean = sum1 / count
    var = sum2 / count - mean * mean
    scale = g * jax.lax.rsqrt(var + eps)
    shift = bt - mean * scale
    return _lane_tile(scale), _lane_tile(shift)


def _pack_diag(wp_ref, w_ref, cout):
    for ky in range(3):
        for kx in range(3):
            blk = w_ref[ky, kx]
            for b in range(B):
                wp_ref[ky, kx, b * C:(b + 1) * C,
                       b * cout:(b + 1) * cout] = blk


def _up2_rows(xa, H, W):
    return jnp.repeat(xa.reshape(H * W, CP), 2, axis=0).reshape(H, 2 * W, CP)


def _up2_into_pad(pa_ref, xa, H, W):
    wu = _up2_rows(xa, H, W)
    for r in range(H):
        row = wu[r]
        pa_ref[1 + 2 * r, 1:2 * W + 1, :] = row
        pa_ref[2 + 2 * r, 1:2 * W + 1, :] = row


def _conv_into(pa_ref, wp_ref, bias_row, y_ref, H, W, R, cin, coutp,
               tanh=False):
    s1 = jnp.zeros((1, coutp), jnp.float32)
    s2 = jnp.zeros((1, coutp), jnp.float32)
    for r0 in range(0, H, R):
        acc = jnp.zeros((R * W, coutp), jnp.float32)
        for dy in range(3):
            for dx in range(3):
                xs = pa_ref[dy + r0:dy + r0 + R, dx:dx + W, :]
                acc = acc + jnp.dot(xs.reshape(R * W, cin), wp_ref[dy, dx],
                                    preferred_element_type=jnp.float32)
        acc = acc + bias_row
        if tanh:
            acc = jnp.tanh(acc)
        y_ref[r0:r0 + R, :, :] = acc.reshape(R, W, coutp)
        s1 = s1 + jnp.sum(acc, axis=0, keepdims=True)
        s2 = s2 + jnp.sum(acc * acc, axis=0, keepdims=True)
    return s1, s2


def _linear_kernel(z_ref, w_ref, b_ref, o_ref):
    acc = jnp.dot(z_ref[...], w_ref[...], preferred_element_type=jnp.float32)
    o_ref[...] = acc + b_ref[0, :][None, :]


def _linear(z, w, b):
    Bz, Z = z.shape
    F = w.shape[1]
    NB = F // 2
    return pl.pallas_call(
        _linear_kernel,
        out_shape=jax.ShapeDtypeStruct((Bz, F), jnp.float32),
        grid=(2,),
        in_specs=[pl.BlockSpec((Bz, Z), lambda i: (0, 0)),
                  pl.BlockSpec((Z, NB), lambda i: (0, i)),
                  pl.BlockSpec((1, NB), lambda i: (0, i))],
        out_specs=pl.BlockSpec((Bz, NB), lambda i: (0, i)),
        compiler_params=pltpu.CompilerParams(
            dimension_semantics=("parallel",)),
    )(z, w, b.reshape(1, F))


def _trunk_kernel(xp_ref, bn0g_ref, bn0b_ref,
                  w1_ref, b1_ref, g1_ref, t1_ref,
                  w2_ref, b2_ref, g2_ref, t2_ref,
                  w3_ref, b3_ref, g3_ref, t3_ref,
                  w4_ref, b4_ref, g4_ref, t4_ref,
                  u5_ref,
                  pa1, pa2, pa3, pa4, y1, y2, y3, y4, wp):
    xp = xp_ref[...]

    flat = xp.reshape(4, CP)
    s1 = jnp.sum(flat, axis=0, keepdims=True)
    s2 = jnp.sum(flat * flat, axis=0, keepdims=True)
    scale_p, shift_p = _bn_from_stats(s1, s2, bn0g_ref[...], bn0b_ref[...],
                                      1e-5, 16.0)

    wp[...] = jnp.zeros_like(wp)

    layers = [(w1_ref, b1_ref, g1_ref, t1_ref, pa1, y1, 4),
              (w2_ref, b2_ref, g2_ref, t2_ref, pa2, y2, 8),
              (w3_ref, b3_ref, g3_ref, t3_ref, pa3, y3, 16),
              (w4_ref, b4_ref, g4_ref, t4_ref, pa4, y4, 8)]

    x_prev = xp
    hw = 2
    for li, (w_ref, b_ref, g_ref, t_ref, pa, y, R) in enumerate(layers):
        a = x_prev * scale_p + shift_p
        if li > 0:
            a = _leaky(a)
        pa[...] = jnp.zeros_like(pa)
        _up2_into_pad(pa, a, hw, hw)
        hw2 = 2 * hw
        _pack_diag(wp, w_ref, C)
        bias = _lane_tile(b_ref[...])
        s1, s2 = _conv_into(pa, wp, bias, y, hw2, hw2, R, CP, CP)
        scale_p, shift_p = _bn_from_stats(s1, s2, g_ref[...], t_ref[...],
                                          0.8, float(B * hw2 * hw2))
        x_prev = y[...]
        hw = hw2

    a = _leaky(x_prev * scale_p + shift_p)
    wu = _up2_rows(a, 32, 32)
    for r in range(32):
        row = wu[r]
        u5_ref[2 * r, :, :] = row
        u5_ref[2 * r + 1, :, :] = row


def _trunk(xp, bn0_g, bn0_b, ws, bs, gs, ts):
    args = [xp, bn0_g.reshape(1, C), bn0_b.reshape(1, C)]
    in_specs = [pl.BlockSpec((2, 2, CP), lambda: (0, 0, 0)),
                pl.BlockSpec((1, C), lambda: (0, 0)),
                pl.BlockSpec((1, C), lambda: (0, 0))]
    for i in range(4):
        args += [ws[i], bs[i].reshape(1, C), gs[i].reshape(1, C),
                 ts[i].reshape(1, C)]
        in_specs += [pl.BlockSpec((3, 3, C, C), lambda: (0, 0, 0, 0)),
                     pl.BlockSpec((1, C), lambda: (0, 0)),
                     pl.BlockSpec((1, C), lambda: (0, 0)),
                     pl.BlockSpec((1, C), lambda: (0, 0))]
    return pl.pallas_call(
        _trunk_kernel,
        out_shape=jax.ShapeDtypeStruct((64, 64, CP), jnp.float32),
        in_specs=in_specs,
        out_specs=pl.BlockSpec((64, 64, CP), lambda: (0, 0, 0)),
        scratch_shapes=[pltpu.VMEM((6, 6, CP), jnp.float32),
                        pltpu.VMEM((10, 10, CP), jnp.float32),
                        pltpu.VMEM((18, 18, CP), jnp.float32),
                        pltpu.VMEM((34, 34, CP), jnp.float32),
                        pltpu.VMEM((4, 4, CP), jnp.float32),
                        pltpu.VMEM((8, 8, CP), jnp.float32),
                        pltpu.VMEM((16, 16, CP), jnp.float32),
                        pltpu.VMEM((32, 32, CP), jnp.float32),
                        pltpu.VMEM((3, 3, CP, CP), jnp.float32)],
    )(*args)


def _conv5_kernel(u_ref, w_ref, b_ref, y_ref, st_ref, pa, wp):
    i = pl.program_id(0)
    r0 = i * 32
    pa[...] = jnp.zeros_like(pa)
    pa[1:33, 1:65, :] = u_ref[pl.ds(r0, 32), :, :]

    @pl.when(i > 0)
    def _():
        pa[0, 1:65, :] = u_ref[r0 - 1, :, :]

    @pl.when(i < pl.num_programs(0) - 1)
    def _():
        pa[33, 1:65, :] = u_ref[r0 + 32, :, :]

    wp[...] = jnp.zeros_like(wp)
    _pack_diag(wp, w_ref, C)
    bias = _lane_tile(b_ref[...])
    s1, s2 = _conv_into(pa, wp, bias, y_ref, 32, 64, 4, CP, CP)
    st_ref[...] = jnp.concatenate([s1, s2], axis=0).reshape(1, 2, CP)


def _conv5(u5, w5, b5):
    return pl.pallas_call(
        _conv5_kernel,
        out_shape=(jax.ShapeDtypeStruct((64, 64, CP), jnp.float32),
                   jax.ShapeDtypeStruct((2, 2, CP), jnp.float32)),
        grid=(2,),
        in_specs=[pl.BlockSpec((64, 64, CP), lambda i: (0, 0, 0)),
                  pl.BlockSpec((3, 3, C, C), lambda i: (0, 0, 0, 0)),
                  pl.BlockSpec((1, C), lambda i: (0, 0))],
        out_specs=(pl.BlockSpec((32, 64, CP), lambda i: (i, 0, 0)),
                   pl.BlockSpec((1, 2, CP), lambda i: (i, 0, 0))),
        scratch_shapes=[pltpu.VMEM((34, 66, CP), jnp.float32),
                        pltpu.VMEM((3, 3, CP, CP), jnp.float32)],
        compiler_params=pltpu.CompilerParams(
            dimension_semantics=("parallel",)),
    )(u5, w5, b5.reshape(1, C))


def _conv6_kernel(y5_ref, st5_ref, g5_ref, t5_ref, w6_ref, b6_ref,
                  t_ref, st_ref, pa, wp):
    i = pl.program_id(0)
    r0 = i * 32

    st5 = st5_ref[...]
    s = st5[0] + st5[1]
    scale_p, shift_p = _bn_from_stats(s[0:1], s[1:2], g5_ref[...],
                                      t5_ref[...], 0.8, float(B * 64 * 64))

    pa[...] = jnp.zeros_like(pa)
    pa[1:33, 1:65, :] = _leaky(
        y5_ref[pl.ds(r0, 32), :, :] * scale_p + shift_p)

    @pl.when(i > 0)
    def _():
        pa[0, 1:65, :] = _leaky(y5_ref[r0 - 1, :, :] * scale_p + shift_p)

    @pl.when(i < pl.num_programs(0) - 1)
    def _():
        pa[33, 1:65, :] = _leaky(y5_ref[r0 + 32, :, :] * scale_p + shift_p)

    wp[...] = jnp.zeros_like(wp)
    _pack_diag(wp, w6_ref, 3)
    bias = jnp.concatenate([b6_ref[...]] * B, axis=1)
    s1, s2 = _conv_into(pa, wp, bias, t_ref, 32, 64, 4, CP, 3 * B, tanh=True)
    st_ref[...] = jnp.concatenate([s1, s2], axis=0).reshape(1, 2, 3 * B)


def _conv6(y5, st5, g5, t5, w6, b6):
    return pl.pallas_call(
        _conv6_kernel,
        out_shape=(jax.ShapeDtypeStruct((64, 64, 3 * B), jnp.float32),
                   jax.ShapeDtypeStruct((2, 2, 3 * B), jnp.float32)),
        grid=(2,),
        in_specs=[pl.BlockSpec((64, 64, CP), lambda i: (0, 0, 0)),
                  pl.BlockSpec((2, 2, CP), lambda i: (0, 0, 0)),
                  pl.BlockSpec((1, C), lambda i: (0, 0)),
                  pl.BlockSpec((1, C), lambda i: (0, 0)),
                  pl.BlockSpec((3, 3, C, 3), lambda i: (0, 0, 0, 0)),
                  pl.BlockSpec((1, 3), lambda i: (0, 0))],
        out_specs=(pl.BlockSpec((32, 64, 3 * B), lambda i: (i, 0, 0)),
                   pl.BlockSpec((1, 2, 3 * B), lambda i: (i, 0, 0))),
        scratch_shapes=[pltpu.VMEM((34, 66, CP), jnp.float32),
                        pltpu.VMEM((3, 3, CP, 3 * B), jnp.float32)],
        compiler_params=pltpu.CompilerParams(
            dimension_semantics=("parallel",)),
    )(y5, st5, g5.reshape(1, C), t5.reshape(1, C), w6, b6.reshape(1, 3))


def kernel(z, l1_w, l1_b, bn0_g, bn0_b,
           conv1_w, conv1_b, bn1_g, bn1_b,
           conv2_w, conv2_b, bn2_g, bn2_b,
           conv3_w, conv3_b, bn3_g, bn3_b,
           conv4_w, conv4_b, bn4_g, bn4_b,
           conv5_w, conv5_b, bn5_g, bn5_b,
           conv6_w, conv6_b):
    lin = _linear(z, l1_w, l1_b)
    xp = jnp.transpose(lin.reshape(B, C, 2, 2), (2, 3, 0, 1)).reshape(2, 2, CP)

    u5 = _trunk(xp, bn0_g, bn0_b,
                [conv1_w, conv2_w, conv3_w, conv4_w],
                [conv1_b, conv2_b, conv3_b, conv4_b],
                [bn1_g, bn2_g, bn3_g, bn4_g],
                [bn1_b, bn2_b, bn3_b, bn4_b])

    y5, st5 = _conv5(u5, conv5_w, conv5_b)
    t, st6 = _conv6(y5, st5, bn5_g, bn5_b, conv6_w, conv6_b)

    s6 = jnp.sum(st6, axis=0)
    sum1 = s6[0].reshape(B, 3).sum(0)
    sum2 = s6[1].reshape(B, 3).sum(0)
    cnt = B * 64 * 64
    mean = sum1 / cnt
    var = sum2 / cnt - mean * mean
    sc = jax.lax.rsqrt(var + 1e-5)
    sh = -mean * sc
    out_p = t * jnp.tile(sc, B) + jnp.tile(sh, B)
    img = out_p.reshape(64, 64, B, 3)
    return jnp.transpose(img, (2, 3, 0, 1))


# speedup vs baseline: 3.6978x; 3.6978x over previous
import jax
import jax.numpy as jnp
from jax.experimental import pallas as pl
from jax.experimental.pallas import tpu as pltpu

B = 4
C = 64
CP = B * C


def _leaky(a):
    return jnp.where(a >= 0, a, 0.2 * a)


def _lane_tile(v):
    return jnp.concatenate([v] * B, axis=1)


def _group_sum(s):
    return sum(s[:, b * C:(b + 1) * C] for b in range(B))


def _bn_from_stats(s1, s2, g, bt, eps, count):
    sum1 = _group_sum(s1)
    sum2 = _group_sum(s2)
    mean = sum1 / count
    var = sum2 / count - mean * mean
    scale = g * jax.lax.rsqrt(var + eps)
    shift = bt - mean * scale
    return _lane_tile(scale), _lane_tile(shift)


def _pack_diag(wp_ref, w_ref, cout):
    for ky in range(3):
        for kx in range(3):
            blk = w_ref[ky, kx]
            for b in range(B):
                wp_ref[ky, kx, b * C:(b + 1) * C,
                       b * cout:(b + 1) * cout] = blk


def _up2_rows(xa, H, W):
    return jnp.repeat(xa.reshape(H * W, CP), 2, axis=0).reshape(H, 2 * W, CP)


def _up2_into_pad(pa_ref, xa, H, W):
    wu = _up2_rows(xa, H, W)
    for r in range(H):
        row = wu[r]
        pa_ref[1 + 2 * r, 1:2 * W + 1, :] = row
        pa_ref[2 + 2 * r, 1:2 * W + 1, :] = row


def _conv_into(pa_ref, wp_ref, bias_row, y_ref, H, W, R, cin, coutp,
               tanh=False):
    s1 = jnp.zeros((1, coutp), jnp.float32)
    s2 = jnp.zeros((1, coutp), jnp.float32)
    for r0 in range(0, H, R):
        acc = jnp.zeros((R * W, coutp), jnp.float32)
        for dy in range(3):
            for dx in range(3):
                xs = pa_ref[dy + r0:dy + r0 + R, dx:dx + W, :]
                acc = acc + jnp.dot(xs.reshape(R * W, cin), wp_ref[dy, dx],
                                    preferred_element_type=jnp.float32)
        acc = acc + bias_row
        if tanh:
            acc = jnp.tanh(acc)
        y_ref[r0:r0 + R, :, :] = acc.reshape(R, W, coutp)
        s1 = s1 + jnp.sum(acc, axis=0, keepdims=True)
        s2 = s2 + jnp.sum(acc * acc, axis=0, keepdims=True)
    return s1, s2


def _linear_kernel(z_ref, w_ref, b_ref, o_ref):
    acc = jnp.dot(z_ref[...], w_ref[...], preferred_element_type=jnp.float32)
    o_ref[...] = acc + b_ref[0, :][None, :]


def _linear(z, w, b):
    Bz, Z = z.shape
    F = w.shape[1]
    NB = F // 2
    return pl.pallas_call(
        _linear_kernel,
        out_shape=jax.ShapeDtypeStruct((Bz, F), jnp.float32),
        grid=(2,),
        in_specs=[pl.BlockSpec((Bz, Z), lambda i: (0, 0)),
                  pl.BlockSpec((Z, NB), lambda i: (0, i)),
                  pl.BlockSpec((1, NB), lambda i: (0, i))],
        out_specs=pl.BlockSpec((Bz, NB), lambda i: (0, i)),
        compiler_params=pltpu.CompilerParams(
            dimension_semantics=("parallel",)),
    )(z, w, b.reshape(1, F))


def _trunk_kernel(xp_ref, bn0g_ref, bn0b_ref,
                  w1_ref, b1_ref, g1_ref, t1_ref,
                  w2_ref, b2_ref, g2_ref, t2_ref,
                  w3_ref, b3_ref, g3_ref, t3_ref,
                  w4_ref, b4_ref, g4_ref, t4_ref,
                  u5_ref,
                  pa1, pa2, pa3, pa4, y1, y2, y3, y4, wp):
    xp = xp_ref[...]

    flat = xp.reshape(4, CP)
    s1 = jnp.sum(flat, axis=0, keepdims=True)
    s2 = jnp.sum(flat * flat, axis=0, keepdims=True)
    scale_p, shift_p = _bn_from_stats(s1, s2, bn0g_ref[...], bn0b_ref[...],
                                      1e-5, 16.0)

    wp[...] = jnp.zeros_like(wp)

    layers = [(w1_ref, b1_ref, g1_ref, t1_ref, pa1, y1, 4),
              (w2_ref, b2_ref, g2_ref, t2_ref, pa2, y2, 8),
              (w3_ref, b3_ref, g3_ref, t3_ref, pa3, y3, 16),
              (w4_ref, b4_ref, g4_ref, t4_ref, pa4, y4, 8)]

    x_prev = xp
    hw = 2
    for li, (w_ref, b_ref, g_ref, t_ref, pa, y, R) in enumerate(layers):
        a = x_prev * scale_p + shift_p
        if li > 0:
            a = _leaky(a)
        pa[...] = jnp.zeros_like(pa)
        _up2_into_pad(pa, a, hw, hw)
        hw2 = 2 * hw
        _pack_diag(wp, w_ref, C)
        bias = _lane_tile(b_ref[...])
        s1, s2 = _conv_into(pa, wp, bias, y, hw2, hw2, R, CP, CP)
        scale_p, shift_p = _bn_from_stats(s1, s2, g_ref[...], t_ref[...],
                                          0.8, float(B * hw2 * hw2))
        x_prev = y[...]
        hw = hw2

    a = _leaky(x_prev * scale_p + shift_p)
    wu = _up2_rows(a, 32, 32)
    for r in range(32):
        row = wu[r]
        u5_ref[2 * r, :, :] = row
        u5_ref[2 * r + 1, :, :] = row


def _trunk(xp, bn0_g, bn0_b, ws, bs, gs, ts):
    args = [xp, bn0_g.reshape(1, C), bn0_b.reshape(1, C)]
    in_specs = [pl.BlockSpec((2, 2, CP), lambda i: (0, 0, 0)),
                pl.BlockSpec((1, C), lambda i: (0, 0)),
                pl.BlockSpec((1, C), lambda i: (0, 0))]
    for _ in range(4):
        args += [ws[_], bs[_].reshape(1, C), gs[_].reshape(1, C),
                 ts[_].reshape(1, C)]
        in_specs += [pl.BlockSpec((3, 3, C, C), lambda i: (0, 0, 0, 0)),
                     pl.BlockSpec((1, C), lambda i: (0, 0)),
                     pl.BlockSpec((1, C), lambda i: (0, 0)),
                     pl.BlockSpec((1, C), lambda i: (0, 0))]
    return pl.pallas_call(
        _trunk_kernel,
        out_shape=jax.ShapeDtypeStruct((64, 64, CP), jnp.float32),
        grid=(1,),
        in_specs=in_specs,
        out_specs=pl.BlockSpec((64, 64, CP), lambda i: (0, 0, 0)),
        scratch_shapes=[pltpu.VMEM((6, 6, CP), jnp.float32),
                        pltpu.VMEM((10, 10, CP), jnp.float32),
                        pltpu.VMEM((18, 18, CP), jnp.float32),
                        pltpu.VMEM((34, 34, CP), jnp.float32),
                        pltpu.VMEM((4, 4, CP), jnp.float32),
                        pltpu.VMEM((8, 8, CP), jnp.float32),
                        pltpu.VMEM((16, 16, CP), jnp.float32),
                        pltpu.VMEM((32, 32, CP), jnp.float32),
                        pltpu.VMEM((3, 3, CP, CP), jnp.float32)],
    )(*args)


def _conv5_kernel(u_ref, w_ref, b_ref, y_ref, st_ref, pa, wp):
    i = pl.program_id(0)
    r0 = i * 32
    pa[...] = jnp.zeros_like(pa)
    pa[1:33, 1:65, :] = u_ref[pl.ds(r0, 32), :, :]

    @pl.when(i > 0)
    def _():
        pa[0, 1:65, :] = u_ref[r0 - 1, :, :]

    @pl.when(i < pl.num_programs(0) - 1)
    def _():
        pa[33, 1:65, :] = u_ref[r0 + 32, :, :]

    wp[...] = jnp.zeros_like(wp)
    _pack_diag(wp, w_ref, C)
    bias = _lane_tile(b_ref[...])
    s1, s2 = _conv_into(pa, wp, bias, y_ref, 32, 64, 4, CP, CP)
    st_ref[...] = jnp.concatenate([s1, s2], axis=0).reshape(1, 2, CP)


def _conv5(u5, w5, b5):
    return pl.pallas_call(
        _conv5_kernel,
        out_shape=(jax.ShapeDtypeStruct((64, 64, CP), jnp.float32),
                   jax.ShapeDtypeStruct((2, 2, CP), jnp.float32)),
        grid=(2,),
        in_specs=[pl.BlockSpec((64, 64, CP), lambda i: (0, 0, 0)),
                  pl.BlockSpec((3, 3, C, C), lambda i: (0, 0, 0, 0)),
                  pl.BlockSpec((1, C), lambda i: (0, 0))],
        out_specs=(pl.BlockSpec((32, 64, CP), lambda i: (i, 0, 0)),
                   pl.BlockSpec((1, 2, CP), lambda i: (i, 0, 0))),
        scratch_shapes=[pltpu.VMEM((34, 66, CP), jnp.float32),
                        pltpu.VMEM((3, 3, CP, CP), jnp.float32)],
        compiler_params=pltpu.CompilerParams(
            dimension_semantics=("parallel",)),
    )(u5, w5, b5.reshape(1, C))


def _conv6_kernel(y5_ref, st5_ref, g5_ref, t5_ref, w6_ref, b6_ref,
                  t_ref, st_ref, pa, wp):
    i = pl.program_id(0)
    r0 = i * 32

    st5 = st5_ref[...]
    s = st5[0] + st5[1]
    scale_p, shift_p = _bn_from_stats(s[0:1], s[1:2], g5_ref[...],
                                      t5_ref[...], 0.8, float(B * 64 * 64))

    pa[...] = jnp.zeros_like(pa)
    pa[1:33, 1:65, :] = _leaky(
        y5_ref[pl.ds(r0, 32), :, :] * scale_p + shift_p)

    @pl.when(i > 0)
    def _():
        pa[0, 1:65, :] = _leaky(y5_ref[r0 - 1, :, :] * scale_p + shift_p)

    @pl.when(i < pl.num_programs(0) - 1)
    def _():
        pa[33, 1:65, :] = _leaky(y5_ref[r0 + 32, :, :] * scale_p + shift_p)

    wp[...] = jnp.zeros_like(wp)
    _pack_diag(wp, w6_ref, 3)
    bias = jnp.concatenate([b6_ref[...]] * B, axis=1)
    s1, s2 = _conv_into(pa, wp, bias, t_ref, 32, 64, 4, CP, 3 * B, tanh=True)
    st_ref[...] = jnp.concatenate([s1, s2], axis=0).reshape(1, 2, 3 * B)


def _conv6(y5, st5, g5, t5, w6, b6):
    return pl.pallas_call(
        _conv6_kernel,
        out_shape=(jax.ShapeDtypeStruct((64, 64, 3 * B), jnp.float32),
                   jax.ShapeDtypeStruct((2, 2, 3 * B), jnp.float32)),
        grid=(2,),
        in_specs=[pl.BlockSpec((64, 64, CP), lambda i: (0, 0, 0)),
                  pl.BlockSpec((2, 2, CP), lambda i: (0, 0, 0)),
                  pl.BlockSpec((1, C), lambda i: (0, 0)),
                  pl.BlockSpec((1, C), lambda i: (0, 0)),
                  pl.BlockSpec((3, 3, C, 3), lambda i: (0, 0, 0, 0)),
                  pl.BlockSpec((1, 3), lambda i: (0, 0))],
        out_specs=(pl.BlockSpec((32, 64, 3 * B), lambda i: (i, 0, 0)),
                   pl.BlockSpec((1, 2, 3 * B), lambda i: (i, 0, 0))),
        scratch_shapes=[pltpu.VMEM((34, 66, CP), jnp.float32),
                        pltpu.VMEM((3, 3, CP, 3 * B), jnp.float32)],
        compiler_params=pltpu.CompilerParams(
            dimension_semantics=("parallel",)),
    )(y5, st5, g5.reshape(1, C), t5.reshape(1, C), w6, b6.reshape(1, 3))


def kernel(z, l1_w, l1_b, bn0_g, bn0_b,
           conv1_w, conv1_b, bn1_g, bn1_b,
           conv2_w, conv2_b, bn2_g, bn2_b,
           conv3_w, conv3_b, bn3_g, bn3_b,
           conv4_w, conv4_b, bn4_g, bn4_b,
           conv5_w, conv5_b, bn5_g, bn5_b,
           conv6_w, conv6_b):
    lin = _linear(z, l1_w, l1_b)
    xp = jnp.transpose(lin.reshape(B, C, 2, 2), (2, 3, 0, 1)).reshape(2, 2, CP)

    u5 = _trunk(xp, bn0_g, bn0_b,
                [conv1_w, conv2_w, conv3_w, conv4_w],
                [conv1_b, conv2_b, conv3_b, conv4_b],
                [bn1_g, bn2_g, bn3_g, bn4_g],
                [bn1_b, bn2_b, bn3_b, bn4_b])

    y5, st5 = _conv5(u5, conv5_w, conv5_b)
    t, st6 = _conv6(y5, st5, bn5_g, bn5_b, conv6_w, conv6_b)

    s6 = jnp.sum(st6, axis=0)
    sum1 = s6[0].reshape(B, 3).sum(0)
    sum2 = s6[1].reshape(B, 3).sum(0)
    cnt = B * 64 * 64
    mean = sum1 / cnt
    var = sum2 / cnt - mean * mean
    sc = jax.lax.rsqrt(var + 1e-5)
    sh = -mean * sc
    out_p = t * jnp.tile(sc, B) + jnp.tile(sh, B)
    img = out_p.reshape(64, 64, B, 3)
    return jnp.transpose(img, (2, 3, 0, 1))


# speedup vs baseline: 5.5995x; 1.5143x over previous
import jax
import jax.numpy as jnp
from jax.experimental import pallas as pl
from jax.experimental.pallas import tpu as pltpu

B = 4
C = 64
CP = B * C


def _leaky(a):
    return jnp.where(a >= 0, a, 0.2 * a)


def _lane_tile(v):
    return jnp.concatenate([v] * B, axis=1)


def _group_sum(s):
    return sum(s[:, b * C:(b + 1) * C] for b in range(B))


def _bn_from_stats(s1, s2, g, bt, eps, count):
    sum1 = _group_sum(s1)
    sum2 = _group_sum(s2)
    mean = sum1 / count
    var = sum2 / count - mean * mean
    scale = g * jax.lax.rsqrt(var + eps)
    shift = bt - mean * scale
    return _lane_tile(scale), _lane_tile(shift)


def _pack_diag(wp_ref, w_ref, cout):
    for ky in range(3):
        for kx in range(3):
            blk = w_ref[ky, kx]
            for b in range(B):
                wp_ref[ky, kx, b * C:(b + 1) * C,
                       b * cout:(b + 1) * cout] = blk


def _up2_rows(xa, H, W):
    return jnp.repeat(xa.reshape(H * W, CP), 2, axis=0).reshape(H, 2 * W, CP)


def _up2_into_pad(pa_ref, xa, H, W):
    wu = _up2_rows(xa, H, W)
    for r in range(H):
        row = wu[r]
        pa_ref[1 + 2 * r, 1:2 * W + 1, :] = row
        pa_ref[2 + 2 * r, 1:2 * W + 1, :] = row


def _conv_into(pa_ref, wp_ref, bias_row, y_ref, H, W, R, cin, coutp,
               tanh=False):
    s1 = jnp.zeros((1, coutp), jnp.float32)
    s2 = jnp.zeros((1, coutp), jnp.float32)
    for r0 in range(0, H, R):
        acc = jnp.zeros((R * W, coutp), jnp.float32)
        for dy in range(3):
            for dx in range(3):
                xs = pa_ref[dy + r0:dy + r0 + R, dx:dx + W, :]
                acc = acc + jnp.dot(xs.reshape(R * W, cin), wp_ref[dy, dx],
                                    preferred_element_type=jnp.float32)
        acc = acc + bias_row
        if tanh:
            acc = jnp.tanh(acc)
        y_ref[r0:r0 + R, :, :] = acc.reshape(R, W, coutp)
        s1 = s1 + jnp.sum(acc, axis=0, keepdims=True)
        s2 = s2 + jnp.sum(acc * acc, axis=0, keepdims=True)
    return s1, s2


def _linear_kernel(z_ref, w_ref, b_ref, o_ref):
    acc = jnp.dot(z_ref[...], w_ref[...], preferred_element_type=jnp.float32)
    o_ref[...] = acc + b_ref[0, :][None, :]


def _linear(z, w, b):
    Bz, Z = z.shape
    F = w.shape[1]
    NB = F // 2
    return pl.pallas_call(
        _linear_kernel,
        out_shape=jax.ShapeDtypeStruct((Bz, F), jnp.float32),
        grid=(2,),
        in_specs=[pl.BlockSpec((Bz, Z), lambda i: (0, 0)),
                  pl.BlockSpec((Z, NB), lambda i: (0, i)),
                  pl.BlockSpec((1, NB), lambda i: (0, i))],
        out_specs=pl.BlockSpec((Bz, NB), lambda i: (0, i)),
        compiler_params=pltpu.CompilerParams(
            dimension_semantics=("parallel",)),
    )(z, w, b.reshape(1, F))


def _net_kernel(lin_ref, bn0g_ref, bn0b_ref,
                w1_ref, b1_ref, g1_ref, t1_ref,
                w2_ref, b2_ref, g2_ref, t2_ref,
                w3_ref, b3_ref, g3_ref, t3_ref,
                w4_ref, b4_ref, g4_ref, t4_ref,
                w5_ref, b5_ref, g5_ref, t5_ref,
                w6_ref, b6_ref,
                out_ref,
                pa1, pa2, pa3, pa4, pa5,
                y1, y2, y3, y4, y5, t6, wp, wp6):
    lin = lin_ref[...]
    xp = jnp.transpose(lin.reshape(B, C, 4), (2, 0, 1)).reshape(2, 2, CP)

    flat = xp.reshape(4, CP)
    s1 = jnp.sum(flat, axis=0, keepdims=True)
    s2 = jnp.sum(flat * flat, axis=0, keepdims=True)
    scale_p, shift_p = _bn_from_stats(s1, s2, bn0g_ref[...], bn0b_ref[...],
                                      1e-5, 16.0)

    wp[...] = jnp.zeros_like(wp)

    layers = [(w1_ref, b1_ref, g1_ref, t1_ref, pa1, y1, 4),
              (w2_ref, b2_ref, g2_ref, t2_ref, pa2, y2, 8),
              (w3_ref, b3_ref, g3_ref, t3_ref, pa3, y3, 16),
              (w4_ref, b4_ref, g4_ref, t4_ref, pa4, y4, 8),
              (w5_ref, b5_ref, g5_ref, t5_ref, pa5, y5, 4)]

    x_prev = xp
    hw = 2
    for li, (w_ref, b_ref, g_ref, t_ref, pa, y, R) in enumerate(layers):
        a = x_prev * scale_p + shift_p
        if li > 0:
            a = _leaky(a)
        pa[...] = jnp.zeros_like(pa)
        _up2_into_pad(pa, a, hw, hw)
        hw2 = 2 * hw
        _pack_diag(wp, w_ref, C)
        bias = _lane_tile(b_ref[...])
        s1, s2 = _conv_into(pa, wp, bias, y, hw2, hw2, R, CP, CP)
        scale_p, shift_p = _bn_from_stats(s1, s2, g_ref[...], t_ref[...],
                                          0.8, float(B * hw2 * hw2))
        x_prev = y[...]
        hw = hw2

    pa5[1:65, 1:65, :] = _leaky(x_prev * scale_p + shift_p)
    wp6[...] = jnp.zeros_like(wp6)
    _pack_diag(wp6, w6_ref, 3)
    bias6 = jnp.concatenate([b6_ref[...]] * B, axis=1)
    s1, s2 = _conv_into(pa5, wp6, bias6, t6, 64, 64, 4, CP, 3 * B, tanh=True)

    sum1 = sum(s1[:, b * 3:(b + 1) * 3] for b in range(B))
    sum2 = sum(s2[:, b * 3:(b + 1) * 3] for b in range(B))
    cnt = float(B * 64 * 64)
    mean = sum1 / cnt
    var = sum2 / cnt - mean * mean
    sc = jax.lax.rsqrt(var + 1e-5)
    sh = -mean * sc
    scale6 = jnp.concatenate([sc] * B, axis=1)
    shift6 = jnp.concatenate([sh] * B, axis=1)
    op = t6[...] * scale6 + shift6
    out_ref[...] = jnp.transpose(op, (2, 0, 1)).reshape(B, 3, 64, 64)


def _net(lin, bn0_g, bn0_b, ws, bs, gs, ts, w6, b6):
    args = [lin, bn0_g.reshape(1, C), bn0_b.reshape(1, C)]
    in_specs = [pl.BlockSpec((B, CP), lambda i: (0, 0)),
                pl.BlockSpec((1, C), lambda i: (0, 0)),
                pl.BlockSpec((1, C), lambda i: (0, 0))]
    for _ in range(5):
        args += [ws[_], bs[_].reshape(1, C), gs[_].reshape(1, C),
                 ts[_].reshape(1, C)]
        in_specs += [pl.BlockSpec((3, 3, C, C), lambda i: (0, 0, 0, 0)),
                     pl.BlockSpec((1, C), lambda i: (0, 0)),
                     pl.BlockSpec((1, C), lambda i: (0, 0)),
                     pl.BlockSpec((1, C), lambda i: (0, 0))]
    args += [w6, b6.reshape(1, 3)]
    in_specs += [pl.BlockSpec((3, 3, C, 3), lambda i: (0, 0, 0, 0)),
                 pl.BlockSpec((1, 3), lambda i: (0, 0))]
    return pl.pallas_call(
        _net_kernel,
        out_shape=jax.ShapeDtypeStruct((B, 3, 64, 64), jnp.float32),
        grid=(1,),
        in_specs=in_specs,
        out_specs=pl.BlockSpec((B, 3, 64, 64), lambda i: (0, 0, 0, 0)),
        scratch_shapes=[pltpu.VMEM((6, 6, CP), jnp.float32),
                        pltpu.VMEM((10, 10, CP), jnp.float32),
                        pltpu.VMEM((18, 18, CP), jnp.float32),
                        pltpu.VMEM((34, 34, CP), jnp.float32),
                        pltpu.VMEM((66, 66, CP), jnp.float32),
                        pltpu.VMEM((4, 4, CP), jnp.float32),
                        pltpu.VMEM((8, 8, CP), jnp.float32),
                        pltpu.VMEM((16, 16, CP), jnp.float32),
                        pltpu.VMEM((32, 32, CP), jnp.float32),
                        pltpu.VMEM((64, 64, CP), jnp.float32),
                        pltpu.VMEM((64, 64, 3 * B), jnp.float32),
                        pltpu.VMEM((3, 3, CP, CP), jnp.float32),
                        pltpu.VMEM((3, 3, CP, 3 * B), jnp.float32)],
    )(*args)


def kernel(z, l1_w, l1_b, bn0_g, bn0_b,
           conv1_w, conv1_b, bn1_g, bn1_b,
           conv2_w, conv2_b, bn2_g, bn2_b,
           conv3_w, conv3_b, bn3_g, bn3_b,
           conv4_w, conv4_b, bn4_g, bn4_b,
           conv5_w, conv5_b, bn5_g, bn5_b,
           conv6_w, conv6_b):
    lin = _linear(z, l1_w, l1_b)
    return _net(lin, bn0_g, bn0_b,
                [conv1_w, conv2_w, conv3_w, conv4_w, conv5_w],
                [conv1_b, conv2_b, conv3_b, conv4_b, conv5_b],
                [bn1_g, bn2_g, bn3_g, bn4_g, bn5_g],
                [bn1_b, bn2_b, bn3_b, bn4_b, bn5_b],
                conv6_w, conv6_b)


# speedup vs baseline: 5.8930x; 1.0524x over previous
import jax
import jax.numpy as jnp
from jax.experimental import pallas as pl
from jax.experimental.pallas import tpu as pltpu

B = 4
C = 64
CP = B * C


def _leaky(a):
    return jnp.where(a >= 0, a, 0.2 * a)


def _lane_tile(v):
    return jnp.concatenate([v] * B, axis=1)


def _group_sum(s):
    return sum(s[:, b * C:(b + 1) * C] for b in range(B))


def _bn_from_stats(s1, s2, g, bt, eps, count):
    sum1 = _group_sum(s1)
    sum2 = _group_sum(s2)
    mean = sum1 / count
    var = sum2 / count - mean * mean
    scale = g * jax.lax.rsqrt(var + eps)
    shift = bt - mean * scale
    return _lane_tile(scale), _lane_tile(shift)


def _pack_diag(wp_ref, w_ref, cout):
    for ky in range(3):
        for kx in range(3):
            blk = w_ref[ky, kx]
            for b in range(B):
                wp_ref[ky, kx, b * C:(b + 1) * C,
                       b * cout:(b + 1) * cout] = blk


def _up2_rows(xa, H, W):
    return jnp.repeat(xa.reshape(H * W, CP), 2, axis=0).reshape(H, 2 * W, CP)


def _up2_into_pad(pa_ref, xa, H, W):
    wu = _up2_rows(xa, H, W)
    for r in range(H):
        row = wu[r]
        pa_ref[1 + 2 * r, 1:2 * W + 1, :] = row
        pa_ref[2 + 2 * r, 1:2 * W + 1, :] = row


def _conv_into(pa_ref, wp_ref, bias_row, y_ref, H, W, R, cin, coutp,
               tanh=False):
    s1 = jnp.zeros((1, coutp), jnp.float32)
    s2 = jnp.zeros((1, coutp), jnp.float32)
    for r0 in range(0, H, R):
        acc = jnp.zeros((R * W, coutp), jnp.float32)
        for dy in range(3):
            for dx in range(3):
                xs = pa_ref[dy + r0:dy + r0 + R, dx:dx + W, :]
                acc = acc + jnp.dot(xs.reshape(R * W, cin), wp_ref[dy, dx],
                                    preferred_element_type=jnp.float32)
        acc = acc + bias_row
        if tanh:
            acc = jnp.tanh(acc)
        y_ref[r0:r0 + R, :, :] = acc.reshape(R, W, coutp)
        s1 = s1 + jnp.sum(acc, axis=0, keepdims=True)
        s2 = s2 + jnp.sum(acc * acc, axis=0, keepdims=True)
    return s1, s2


_KT = 4096
_NK = 16384 // _KT


def _net_kernel(z_ref, lw_ref, lb_ref, bn0g_ref, bn0b_ref,
                w1_ref, b1_ref, g1_ref, t1_ref,
                w2_ref, b2_ref, g2_ref, t2_ref,
                w3_ref, b3_ref, g3_ref, t3_ref,
                w4_ref, b4_ref, g4_ref, t4_ref,
                w5_ref, b5_ref, g5_ref, t5_ref,
                w6_ref, b6_ref,
                out_ref,
                acc,
                pa1, pa2, pa3, pa4, pa5,
                y1, y2, y3, y4, y5, t6, wp, wp6):
    k = pl.program_id(0)

    @pl.when(k == 0)
    def _():
        acc[...] = jnp.zeros_like(acc)

    @pl.when(k < _NK)
    def _():
        acc[...] = acc[...] + jnp.dot(z_ref[...], lw_ref[...],
                                      preferred_element_type=jnp.float32)

    @pl.when(k == _NK)
    def _():
        _net_body(acc, lb_ref, bn0g_ref, bn0b_ref,
                  w1_ref, b1_ref, g1_ref, t1_ref,
                  w2_ref, b2_ref, g2_ref, t2_ref,
                  w3_ref, b3_ref, g3_ref, t3_ref,
                  w4_ref, b4_ref, g4_ref, t4_ref,
                  w5_ref, b5_ref, g5_ref, t5_ref,
                  w6_ref, b6_ref, out_ref,
                  pa1, pa2, pa3, pa4, pa5,
                  y1, y2, y3, y4, y5, t6, wp, wp6)


def _net_body(acc, lb_ref, bn0g_ref, bn0b_ref,
              w1_ref, b1_ref, g1_ref, t1_ref,
              w2_ref, b2_ref, g2_ref, t2_ref,
              w3_ref, b3_ref, g3_ref, t3_ref,
              w4_ref, b4_ref, g4_ref, t4_ref,
              w5_ref, b5_ref, g5_ref, t5_ref,
              w6_ref, b6_ref, out_ref,
              pa1, pa2, pa3, pa4, pa5,
              y1, y2, y3, y4, y5, t6, wp, wp6):
    lin = acc[...] + lb_ref[...]
    xp = jnp.transpose(lin.reshape(B, C, 4), (2, 0, 1)).reshape(2, 2, CP)

    flat = xp.reshape(4, CP)
    s1 = jnp.sum(flat, axis=0, keepdims=True)
    s2 = jnp.sum(flat * flat, axis=0, keepdims=True)
    scale_p, shift_p = _bn_from_stats(s1, s2, bn0g_ref[...], bn0b_ref[...],
                                      1e-5, 16.0)

    wp[...] = jnp.zeros_like(wp)

    layers = [(w1_ref, b1_ref, g1_ref, t1_ref, pa1, y1, 4),
              (w2_ref, b2_ref, g2_ref, t2_ref, pa2, y2, 8),
              (w3_ref, b3_ref, g3_ref, t3_ref, pa3, y3, 16),
              (w4_ref, b4_ref, g4_ref, t4_ref, pa4, y4, 8),
              (w5_ref, b5_ref, g5_ref, t5_ref, pa5, y5, 4)]

    x_prev = xp
    hw = 2
    for li, (w_ref, b_ref, g_ref, t_ref, pa, y, R) in enumerate(layers):
        a = x_prev * scale_p + shift_p
        if li > 0:
            a = _leaky(a)
        pa[...] = jnp.zeros_like(pa)
        _up2_into_pad(pa, a, hw, hw)
        hw2 = 2 * hw
        _pack_diag(wp, w_ref, C)
        bias = _lane_tile(b_ref[...])
        s1, s2 = _conv_into(pa, wp, bias, y, hw2, hw2, R, CP, CP)
        scale_p, shift_p = _bn_from_stats(s1, s2, g_ref[...], t_ref[...],
                                          0.8, float(B * hw2 * hw2))
        x_prev = y[...]
        hw = hw2

    pa5[1:65, 1:65, :] = _leaky(x_prev * scale_p + shift_p)
    wp6[...] = jnp.zeros_like(wp6)
    _pack_diag(wp6, w6_ref, 3)
    bias6 = jnp.concatenate([b6_ref[...]] * B, axis=1)
    s1, s2 = _conv_into(pa5, wp6, bias6, t6, 64, 64, 4, CP, 3 * B, tanh=True)

    sum1 = sum(s1[:, b * 3:(b + 1) * 3] for b in range(B))
    sum2 = sum(s2[:, b * 3:(b + 1) * 3] for b in range(B))
    cnt = float(B * 64 * 64)
    mean = sum1 / cnt
    var = sum2 / cnt - mean * mean
    sc = jax.lax.rsqrt(var + 1e-5)
    sh = -mean * sc
    scale6 = jnp.concatenate([sc] * B, axis=1)
    shift6 = jnp.concatenate([sh] * B, axis=1)
    op = t6[...] * scale6 + shift6
    out_ref[...] = jnp.transpose(op, (2, 0, 1)).reshape(B, 3, 64, 64)


def _net(z, l1_w, l1_b, bn0_g, bn0_b, ws, bs, gs, ts, w6, b6):
    last = _NK - 1
    args = [z, l1_w, l1_b.reshape(1, CP),
            bn0_g.reshape(1, C), bn0_b.reshape(1, C)]
    in_specs = [pl.BlockSpec((B, _KT), lambda i: (0, jnp.minimum(i, last))),
                pl.BlockSpec((_KT, CP), lambda i: (jnp.minimum(i, last), 0)),
                pl.BlockSpec((1, CP), lambda i: (0, 0)),
                pl.BlockSpec((1, C), lambda i: (0, 0)),
                pl.BlockSpec((1, C), lambda i: (0, 0))]
    for _ in range(5):
        args += [ws[_], bs[_].reshape(1, C), gs[_].reshape(1, C),
                 ts[_].reshape(1, C)]
        in_specs += [pl.BlockSpec((3, 3, C, C), lambda i: (0, 0, 0, 0)),
                     pl.BlockSpec((1, C), lambda i: (0, 0)),
                     pl.BlockSpec((1, C), lambda i: (0, 0)),
                     pl.BlockSpec((1, C), lambda i: (0, 0))]
    args += [w6, b6.reshape(1, 3)]
    in_specs += [pl.BlockSpec((3, 3, C, 3), lambda i: (0, 0, 0, 0)),
                 pl.BlockSpec((1, 3), lambda i: (0, 0))]
    return pl.pallas_call(
        _net_kernel,
        out_shape=jax.ShapeDtypeStruct((B, 3, 64, 64), jnp.float32),
        grid=(_NK + 1,),
        in_specs=in_specs,
        out_specs=pl.BlockSpec((B, 3, 64, 64), lambda i: (0, 0, 0, 0)),
        compiler_params=pltpu.CompilerParams(
            dimension_semantics=("arbitrary",)),
        scratch_shapes=[pltpu.VMEM((B, CP), jnp.float32),
                        pltpu.VMEM((6, 6, CP), jnp.float32),
                        pltpu.VMEM((10, 10, CP), jnp.float32),
                        pltpu.VMEM((18, 18, CP), jnp.float32),
                        pltpu.VMEM((34, 34, CP), jnp.float32),
                        pltpu.VMEM((66, 66, CP), jnp.float32),
                        pltpu.VMEM((4, 4, CP), jnp.float32),
                        pltpu.VMEM((8, 8, CP), jnp.float32),
                        pltpu.VMEM((16, 16, CP), jnp.float32),
                        pltpu.VMEM((32, 32, CP), jnp.float32),
                        pltpu.VMEM((64, 64, CP), jnp.float32),
                        pltpu.VMEM((64, 64, 3 * B), jnp.float32),
                        pltpu.VMEM((3, 3, CP, CP), jnp.float32),
                        pltpu.VMEM((3, 3, CP, 3 * B), jnp.float32)],
    )(*args)


def kernel(z, l1_w, l1_b, bn0_g, bn0_b,
           conv1_w, conv1_b, bn1_g, bn1_b,
           conv2_w, conv2_b, bn2_g, bn2_b,
           conv3_w, conv3_b, bn3_g, bn3_b,
           conv4_w, conv4_b, bn4_g, bn4_b,
           conv5_w, conv5_b, bn5_g, bn5_b,
           conv6_w, conv6_b):
    return _net(z, l1_w, l1_b, bn0_g, bn0_b,
                [conv1_w, conv2_w, conv3_w, conv4_w, conv5_w],
                [conv1_b, conv2_b, conv3_b, conv4_b, conv5_b],
                [bn1_g, bn2_g, bn3_g, bn4_g, bn5_g],
                [bn1_b, bn2_b, bn3_b, bn4_b, bn5_b],
                conv6_w, conv6_b)


# speedup vs baseline: 6.3680x; 1.0806x over previous
import jax
import jax.numpy as jnp
from jax.experimental import pallas as pl
from jax.experimental.pallas import tpu as pltpu

B = 4
C = 64
CP = B * C


def _leaky(a):
    return jnp.where(a >= 0, a, 0.2 * a)


def _lane_tile(v):
    return jnp.concatenate([v] * B, axis=1)


def _group_sum(s):
    return sum(s[:, b * C:(b + 1) * C] for b in range(B))


def _bn_from_stats(s1, s2, g, bt, eps, count):
    sum1 = _group_sum(s1)
    sum2 = _group_sum(s2)
    mean = sum1 / count
    var = sum2 / count - mean * mean
    scale = g * jax.lax.rsqrt(var + eps)
    shift = bt - mean * scale
    return _lane_tile(scale), _lane_tile(shift)


def _pack_diag3(wp_ref, w_ref, cout):
    for ky in range(3):
        for kx in range(3):
            blk = w_ref[ky, kx]
            for b in range(B):
                wp_ref[ky, kx * CP + b * C:kx * CP + (b + 1) * C,
                       b * cout:(b + 1) * cout] = blk


def _up2_rows(xa, H, W):
    return jnp.repeat(xa.reshape(H * W, CP), 2, axis=0).reshape(H, 2 * W, CP)


def _tri_lanes(rows, W):
    H = rows.shape[0]
    z1 = jnp.zeros((H, 1, CP), jnp.float32)
    up = jnp.concatenate([z1, rows, z1], axis=1)
    return jnp.concatenate(
        [up[:, 0:W], up[:, 1:W + 1], up[:, 2:W + 2]], axis=2)


def _zero_row(pa_ref, r):
    pa_ref[r, :, :] = jnp.zeros(pa_ref.shape[1:], jnp.float32)


def _up2_into_pad(pa_ref, xa, H, W):
    tri = _tri_lanes(_up2_rows(xa, H, W), 2 * W)
    _zero_row(pa_ref, 0)
    _zero_row(pa_ref, 2 * H + 1)
    for r in range(H):
        row = tri[r]
        pa_ref[1 + 2 * r, :, :] = row
        pa_ref[2 + 2 * r, :, :] = row


def _conv_into(pa_ref, wp_ref, bias_row, y_ref, H, W, R, coutp, tanh=False):
    s1 = jnp.zeros((1, coutp), jnp.float32)
    s2 = jnp.zeros((1, coutp), jnp.float32)
    for r0 in range(0, H, R):
        acc = jnp.zeros((R * W, coutp), jnp.float32)
        for dy in range(3):
            xs = pa_ref[dy + r0:dy + r0 + R, :, :]
            acc = acc + jnp.dot(xs.reshape(R * W, 3 * CP), wp_ref[dy],
                                preferred_element_type=jnp.float32)
        acc = acc + bias_row
        if tanh:
            acc = jnp.tanh(acc)
        y_ref[r0:r0 + R, :, :] = acc.reshape(R, W, coutp)
        s1 = s1 + jnp.sum(acc, axis=0, keepdims=True)
        s2 = s2 + jnp.sum(acc * acc, axis=0, keepdims=True)
    return s1, s2


_KT = 4096
_NK = 16384 // _KT


def _net_kernel(z_ref, lw_ref, lb_ref, bn0g_ref, bn0b_ref,
                w1_ref, b1_ref, g1_ref, t1_ref,
                w2_ref, b2_ref, g2_ref, t2_ref,
                w3_ref, b3_ref, g3_ref, t3_ref,
                w4_ref, b4_ref, g4_ref, t4_ref,
                w5_ref, b5_ref, g5_ref, t5_ref,
                w6_ref, b6_ref,
                out_ref,
                acc,
                pa1, pa2, pa3, pa4, pa5,
                y1, y2, y3, y4, y5, t6, wp, wp6):
    k = pl.program_id(0)

    @pl.when(k == 0)
    def _():
        acc[...] = jnp.zeros_like(acc)

    @pl.when(k < _NK)
    def _():
        acc[...] = acc[...] + jnp.dot(z_ref[...], lw_ref[...],
                                      preferred_element_type=jnp.float32)

    @pl.when(k == _NK)
    def _():
        _net_body(acc, lb_ref, bn0g_ref, bn0b_ref,
                  w1_ref, b1_ref, g1_ref, t1_ref,
                  w2_ref, b2_ref, g2_ref, t2_ref,
                  w3_ref, b3_ref, g3_ref, t3_ref,
                  w4_ref, b4_ref, g4_ref, t4_ref,
                  w5_ref, b5_ref, g5_ref, t5_ref,
                  w6_ref, b6_ref, out_ref,
                  pa1, pa2, pa3, pa4, pa5,
                  y1, y2, y3, y4, y5, t6, wp, wp6)


def _net_body(acc, lb_ref, bn0g_ref, bn0b_ref,
              w1_ref, b1_ref, g1_ref, t1_ref,
              w2_ref, b2_ref, g2_ref, t2_ref,
              w3_ref, b3_ref, g3_ref, t3_ref,
              w4_ref, b4_ref, g4_ref, t4_ref,
              w5_ref, b5_ref, g5_ref, t5_ref,
              w6_ref, b6_ref, out_ref,
              pa1, pa2, pa3, pa4, pa5,
              y1, y2, y3, y4, y5, t6, wp, wp6):
    lin = acc[...] + lb_ref[...]
    xp = jnp.transpose(lin.reshape(B, C, 4), (2, 0, 1)).reshape(2, 2, CP)

    flat = xp.reshape(4, CP)
    s1 = jnp.sum(flat, axis=0, keepdims=True)
    s2 = jnp.sum(flat * flat, axis=0, keepdims=True)
    scale_p, shift_p = _bn_from_stats(s1, s2, bn0g_ref[...], bn0b_ref[...],
                                      1e-5, 16.0)

    wp[...] = jnp.zeros_like(wp)

    layers = [(w1_ref, b1_ref, g1_ref, t1_ref, pa1, y1, 4),
              (w2_ref, b2_ref, g2_ref, t2_ref, pa2, y2, 8),
              (w3_ref, b3_ref, g3_ref, t3_ref, pa3, y3, 16),
              (w4_ref, b4_ref, g4_ref, t4_ref, pa4, y4, 8),
              (w5_ref, b5_ref, g5_ref, t5_ref, pa5, y5, 4)]

    x_prev = xp
    hw = 2
    for li, (w_ref, b_ref, g_ref, t_ref, pa, y, R) in enumerate(layers):
        a = x_prev * scale_p + shift_p
        if li > 0:
            a = _leaky(a)
        _up2_into_pad(pa, a, hw, hw)
        hw2 = 2 * hw
        _pack_diag3(wp, w_ref, C)
        bias = _lane_tile(b_ref[...])
        s1, s2 = _conv_into(pa, wp, bias, y, hw2, hw2, R, CP)
        scale_p, shift_p = _bn_from_stats(s1, s2, g_ref[...], t_ref[...],
                                          0.8, float(B * hw2 * hw2))
        x_prev = y[...]
        hw = hw2

    pa5[1:65, :, :] = _tri_lanes(_leaky(x_prev * scale_p + shift_p), 64)
    wp6[...] = jnp.zeros_like(wp6)
    _pack_diag3(wp6, w6_ref, 3)
    bias6 = jnp.concatenate([b6_ref[...]] * B, axis=1)
    s1, s2 = _conv_into(pa5, wp6, bias6, t6, 64, 64, 4, 3 * B, tanh=True)

    sum1 = sum(s1[:, b * 3:(b + 1) * 3] for b in range(B))
    sum2 = sum(s2[:, b * 3:(b + 1) * 3] for b in range(B))
    cnt = float(B * 64 * 64)
    mean = sum1 / cnt
    var = sum2 / cnt - mean * mean
    sc = jax.lax.rsqrt(var + 1e-5)
    sh = -mean * sc
    scale6 = jnp.concatenate([sc] * B, axis=1)
    shift6 = jnp.concatenate([sh] * B, axis=1)
    op = t6[...] * scale6 + shift6
    out_ref[...] = jnp.transpose(op, (2, 0, 1)).reshape(B, 3, 64, 64)


def _net(z, l1_w, l1_b, bn0_g, bn0_b, ws, bs, gs, ts, w6, b6):
    last = _NK - 1
    args = [z, l1_w, l1_b.reshape(1, CP),
            bn0_g.reshape(1, C), bn0_b.reshape(1, C)]
    in_specs = [pl.BlockSpec((B, _KT), lambda i: (0, jnp.minimum(i, last))),
                pl.BlockSpec((_KT, CP), lambda i: (jnp.minimum(i, last), 0)),
                pl.BlockSpec((1, CP), lambda i: (0, 0)),
                pl.BlockSpec((1, C), lambda i: (0, 0)),
                pl.BlockSpec((1, C), lambda i: (0, 0))]
    for _ in range(5):
        args += [ws[_], bs[_].reshape(1, C), gs[_].reshape(1, C),
                 ts[_].reshape(1, C)]
        in_specs += [pl.BlockSpec((3, 3, C, C), lambda i: (0, 0, 0, 0)),
                     pl.BlockSpec((1, C), lambda i: (0, 0)),
                     pl.BlockSpec((1, C), lambda i: (0, 0)),
                     pl.BlockSpec((1, C), lambda i: (0, 0))]
    args += [w6, b6.reshape(1, 3)]
    in_specs += [pl.BlockSpec((3, 3, C, 3), lambda i: (0, 0, 0, 0)),
                 pl.BlockSpec((1, 3), lambda i: (0, 0))]
    return pl.pallas_call(
        _net_kernel,
        out_shape=jax.ShapeDtypeStruct((B, 3, 64, 64), jnp.float32),
        grid=(_NK + 1,),
        in_specs=in_specs,
        out_specs=pl.BlockSpec((B, 3, 64, 64), lambda i: (0, 0, 0, 0)),
        compiler_params=pltpu.CompilerParams(
            dimension_semantics=("arbitrary",)),
        scratch_shapes=[pltpu.VMEM((B, CP), jnp.float32),
                        pltpu.VMEM((6, 4, 3 * CP), jnp.float32),
                        pltpu.VMEM((10, 8, 3 * CP), jnp.float32),
                        pltpu.VMEM((18, 16, 3 * CP), jnp.float32),
                        pltpu.VMEM((34, 32, 3 * CP), jnp.float32),
                        pltpu.VMEM((66, 64, 3 * CP), jnp.float32),
                        pltpu.VMEM((4, 4, CP), jnp.float32),
                        pltpu.VMEM((8, 8, CP), jnp.float32),
                        pltpu.VMEM((16, 16, CP), jnp.float32),
                        pltpu.VMEM((32, 32, CP), jnp.float32),
                        pltpu.VMEM((64, 64, CP), jnp.float32),
                        pltpu.VMEM((64, 64, 3 * B), jnp.float32),
                        pltpu.VMEM((3, 3 * CP, CP), jnp.float32),
                        pltpu.VMEM((3, 3 * CP, 3 * B), jnp.float32)],
    )(*args)


def kernel(z, l1_w, l1_b, bn0_g, bn0_b,
           conv1_w, conv1_b, bn1_g, bn1_b,
           conv2_w, conv2_b, bn2_g, bn2_b,
           conv3_w, conv3_b, bn3_g, bn3_b,
           conv4_w, conv4_b, bn4_g, bn4_b,
           conv5_w, conv5_b, bn5_g, bn5_b,
           conv6_w, conv6_b):
    return _net(z, l1_w, l1_b, bn0_g, bn0_b,
                [conv1_w, conv2_w, conv3_w, conv4_w, conv5_w],
                [conv1_b, conv2_b, conv3_b, conv4_b, conv5_b],
                [bn1_g, bn2_g, bn3_g, bn4_g, bn5_g],
                [bn1_b, bn2_b, bn3_b, bn4_b, bn5_b],
                conv6_w, conv6_b)
